# baseline (speedup 1.0000x reference)
import functools

import jax
import jax.numpy as jnp
from jax import lax
from jax.experimental import pallas as pl
from jax.experimental.pallas import tpu as pltpu
from jax.experimental.pallas import tpu_sc as plsc

_CHUNK_ELEMS = 32 * 1024


def kernel(x, weight):
    batch, seq, dim = x.shape
    total = batch * seq * dim
    wtotal = seq * dim
    info = plsc.get_sparse_core_info()
    nw = info.num_cores * info.num_subcores
    per_worker = total // nw
    chunk = _CHUNK_ELEMS
    n_chunks = per_worker // chunk
    assert per_worker % chunk == 0 and wtotal % per_worker == 0

    mesh = plsc.VectorSubcoreMesh(core_axis_name="c", subcore_axis_name="s")

    @functools.partial(
        pl.kernel,
        mesh=mesh,
        out_type=jax.ShapeDtypeStruct((total,), jnp.float32),
        scratch_types=[
            pltpu.VMEM((chunk,), jnp.float32),
            pltpu.VMEM((chunk,), jnp.float32),
        ],
    )
    def sc_add(x_hbm, w_hbm, out_hbm, xb, wb):
        wid = lax.axis_index("s") * info.num_cores + lax.axis_index("c")
        base = wid * per_worker
        wbase = lax.rem(base, wtotal)

        def step(i, _):
            off = base + i * chunk
            woff = wbase + i * chunk
            pltpu.sync_copy(x_hbm.at[pl.ds(off, chunk)], xb)
            pltpu.sync_copy(w_hbm.at[pl.ds(woff, chunk)], wb)

            @plsc.parallel_loop(0, chunk // 16, unroll=8)
            def add_body(k):
                s = pl.ds(k * 16, 16)
                xb[s] = xb[s] + wb[s]

            pltpu.sync_copy(xb, out_hbm.at[pl.ds(off, chunk)])
            return 0

        lax.fori_loop(0, n_chunks, step, 0)

    out = sc_add(x.reshape(total), weight.reshape(wtotal))
    return out.reshape(batch, seq, dim)


# speedup vs baseline: 1.2509x; 1.2509x over previous
import functools

import jax
import jax.numpy as jnp
from jax import lax
from jax.experimental import pallas as pl
from jax.experimental.pallas import tpu as pltpu
from jax.experimental.pallas import tpu_sc as plsc

_CHUNK_ELEMS = 16 * 1024


def kernel(x, weight):
    batch, seq, dim = x.shape
    total = batch * seq * dim
    wtotal = seq * dim
    info = plsc.get_sparse_core_info()
    nw = info.num_cores * info.num_subcores
    per_worker = total // nw
    chunk = _CHUNK_ELEMS
    n_chunks = per_worker // chunk
    assert per_worker % chunk == 0 and wtotal % per_worker == 0
    assert n_chunks % 2 == 0

    mesh = plsc.VectorSubcoreMesh(core_axis_name="c", subcore_axis_name="s")

    @functools.partial(
        pl.kernel,
        mesh=mesh,
        out_type=jax.ShapeDtypeStruct((total,), jnp.float32),
        scratch_types=[
            pltpu.VMEM((chunk,), jnp.float32),
            pltpu.VMEM((chunk,), jnp.float32),
            pltpu.VMEM((chunk,), jnp.float32),
            pltpu.VMEM((chunk,), jnp.float32),
            pltpu.VMEM((chunk,), jnp.float32),
            pltpu.VMEM((chunk,), jnp.float32),
            pltpu.SemaphoreType.DMA,
            pltpu.SemaphoreType.DMA,
        ],
    )
    def sc_add(x_hbm, w_hbm, out_hbm, xb0, xb1, wb0, wb1, yb0, yb1, in_sem, out_sem):
        wid = lax.axis_index("s") * info.num_cores + lax.axis_index("c")
        base = wid * per_worker
        wbase = lax.rem(base, wtotal)
        xbufs = (xb0, xb1)
        wbufs = (wb0, wb1)
        ybufs = (yb0, yb1)

        def start_in(i, s):
            pltpu.async_copy(x_hbm.at[pl.ds(base + i * chunk, chunk)], xbufs[s], in_sem)
            pltpu.async_copy(w_hbm.at[pl.ds(wbase + i * chunk, chunk)], wbufs[s], in_sem)

        def wait_in(i, s):
            pltpu.make_async_copy(x_hbm.at[pl.ds(base + i * chunk, chunk)], xbufs[s], in_sem).wait()
            pltpu.make_async_copy(w_hbm.at[pl.ds(wbase + i * chunk, chunk)], wbufs[s], in_sem).wait()

        def start_out(i, s):
            pltpu.async_copy(ybufs[s], out_hbm.at[pl.ds(base + i * chunk, chunk)], out_sem)

        def wait_out(i, s):
            pltpu.make_async_copy(ybufs[s], out_hbm.at[pl.ds(base + i * chunk, chunk)], out_sem).wait()

        start_in(0, 0)
        start_in(1, 1)

        def pair(p, _):
            for s in (0, 1):
                i = 2 * p + s
                wait_in(i, s)

                @pl.when(p >= 1)
                def _():
                    wait_out(i - 2, s)

                xb, wb, yb = xbufs[s], wbufs[s], ybufs[s]

                @plsc.parallel_loop(0, chunk // 16, unroll=16)
                def add_body(k):
                    sl = pl.ds(k * 16, 16)
                    yb[sl] = xb[sl] + wb[sl]

                @pl.when(i + 2 < n_chunks)
                def _():
                    start_in(i + 2, s)

                start_out(i, s)
            return 0

        lax.fori_loop(0, n_chunks // 2, pair, 0)
        wait_out(n_chunks - 2, 0)
        wait_out(n_chunks - 1, 1)

    out = sc_add(x.reshape(total), weight.reshape(wtotal))
    return out.reshape(batch, seq, dim)


# speedup vs baseline: 1.8398x; 1.4707x over previous
import functools

import jax
import jax.numpy as jnp
from jax import lax
from jax.experimental import pallas as pl
from jax.experimental.pallas import tpu as pltpu
from jax.experimental.pallas import tpu_sc as plsc

_SEQ_BLOCK = 512
_TAIL_ROWS = 1536
_CHUNK_ELEMS = 16 * 1024


def _add_kernel(x_ref, w_ref, o_ref):
    o_ref[...] = x_ref[...] + w_ref[...][None, :, :]


def _tc_head(x, weight, head):
    batch, seq, dim = x.shape
    blk = _SEQ_BLOCK
    return pl.pallas_call(
        _add_kernel,
        grid=(head // blk,),
        in_specs=[
            pl.BlockSpec((batch, blk, dim), lambda i: (0, i, 0)),
            pl.BlockSpec((blk, dim), lambda i: (i, 0)),
        ],
        out_specs=pl.BlockSpec((batch, blk, dim), lambda i: (0, i, 0)),
        out_shape=jax.ShapeDtypeStruct((batch, seq, dim), x.dtype),
    )(x, weight)


def _sc_tail(x, weight, head, tail):
    batch, seq, dim = x.shape
    info = plsc.get_sparse_core_info()
    nw = info.num_cores * info.num_subcores
    out_total = batch * tail * dim
    per_worker = out_total // nw
    rows_per_worker = tail // (nw // batch)
    chunk = _CHUNK_ELEMS
    n_chunks = per_worker // chunk
    assert per_worker % chunk == 0 and n_chunks % 2 == 0
    assert tail % (nw // batch) == 0 and nw % batch == 0
    wpb = nw // batch

    mesh = plsc.VectorSubcoreMesh(core_axis_name="c", subcore_axis_name="s")

    @functools.partial(
        pl.kernel,
        mesh=mesh,
        out_type=jax.ShapeDtypeStruct((out_total,), jnp.float32),
        scratch_types=[
            pltpu.VMEM((chunk,), jnp.float32),
            pltpu.VMEM((chunk,), jnp.float32),
            pltpu.VMEM((chunk,), jnp.float32),
            pltpu.VMEM((chunk,), jnp.float32),
            pltpu.VMEM((chunk,), jnp.float32),
            pltpu.VMEM((chunk,), jnp.float32),
            pltpu.SemaphoreType.DMA,
            pltpu.SemaphoreType.DMA,
        ],
    )
    def sc_add(x_hbm, w_hbm, out_hbm, xb0, xb1, wb0, wb1, yb0, yb1, in_sem, out_sem):
        wid = lax.axis_index("s") * info.num_cores + lax.axis_index("c")
        b = wid // wpb
        r0 = head + (wid % wpb) * rows_per_worker
        xbase = (b * seq + r0) * dim
        wbase = r0 * dim
        obase = wid * per_worker
        xbufs = (xb0, xb1)
        wbufs = (wb0, wb1)
        ybufs = (yb0, yb1)

        def start_in(i, s):
            pltpu.async_copy(x_hbm.at[pl.ds(xbase + i * chunk, chunk)], xbufs[s], in_sem)
            pltpu.async_copy(w_hbm.at[pl.ds(wbase + i * chunk, chunk)], wbufs[s], in_sem)

        def wait_in(i, s):
            pltpu.make_async_copy(x_hbm.at[pl.ds(xbase + i * chunk, chunk)], xbufs[s], in_sem).wait()
            pltpu.make_async_copy(w_hbm.at[pl.ds(wbase + i * chunk, chunk)], wbufs[s], in_sem).wait()

        def start_out(i, s):
            pltpu.async_copy(ybufs[s], out_hbm.at[pl.ds(obase + i * chunk, chunk)], out_sem)

        def wait_out(i, s):
            pltpu.make_async_copy(ybufs[s], out_hbm.at[pl.ds(obase + i * chunk, chunk)], out_sem).wait()

        start_in(0, 0)
        start_in(1, 1)

        def pair(p, _):
            for s in (0, 1):
                i = 2 * p + s
                wait_in(i, s)

                @pl.when(p >= 1)
                def _():
                    wait_out(i - 2, s)

                xb, wb, yb = xbufs[s], wbufs[s], ybufs[s]

                @plsc.parallel_loop(0, chunk // 16, unroll=16)
                def add_body(k):
                    sl = pl.ds(k * 16, 16)
                    yb[sl] = xb[sl] + wb[sl]

                @pl.when(i + 2 < n_chunks)
                def _():
                    start_in(i + 2, s)

                start_out(i, s)
            return 0

        lax.fori_loop(0, n_chunks // 2, pair, 0)
        wait_out(n_chunks - 2, 0)
        wait_out(n_chunks - 1, 1)

    out = sc_add(x.reshape(batch * seq * dim), weight.reshape(seq * dim))
    return out.reshape(batch, tail, dim)


def kernel(x, weight):
    batch, seq, dim = x.shape
    tail = _TAIL_ROWS
    head = seq - tail
    sc_part = _sc_tail(x, weight, head, tail)
    tc_out = _tc_head(x, weight, head)
    return lax.dynamic_update_slice(tc_out, sc_part, (0, head, 0))


# speedup vs baseline: 5.6038x; 3.0459x over previous
import jax
import jax.numpy as jnp
from jax.experimental import pallas as pl

_SEQ_BLOCK = 256


def _add_kernel(x_ref, w_ref, o_ref):
    o_ref[...] = x_ref[...] + w_ref[...][None, :, :]


def kernel(x, weight):
    batch, seq, dim = x.shape
    w = weight[:seq, :]
    blk = _SEQ_BLOCK if seq % _SEQ_BLOCK == 0 else seq
    grid = (seq // blk,)
    return pl.pallas_call(
        _add_kernel,
        grid=grid,
        in_specs=[
            pl.BlockSpec((batch, blk, dim), lambda i: (0, i, 0)),
            pl.BlockSpec((blk, dim), lambda i: (i, 0)),
        ],
        out_specs=pl.BlockSpec((batch, blk, dim), lambda i: (0, i, 0)),
        out_shape=jax.ShapeDtypeStruct((batch, seq, dim), x.dtype),
    )(x, w)


# speedup vs baseline: 5.6341x; 1.0054x over previous
import jax
import jax.numpy as jnp
from jax.experimental import pallas as pl

_SEQ_BLOCK = 512


def _add_kernel(x_ref, w_ref, o_ref):
    o_ref[...] = x_ref[...] + w_ref[...][None, :, :]


def kernel(x, weight):
    batch, seq, dim = x.shape
    w = weight[:seq, :]
    blk = _SEQ_BLOCK if seq % _SEQ_BLOCK == 0 else seq
    grid = (seq // blk,)
    return pl.pallas_call(
        _add_kernel,
        grid=grid,
        in_specs=[
            pl.BlockSpec((batch, blk, dim), lambda i: (0, i, 0)),
            pl.BlockSpec((blk, dim), lambda i: (i, 0)),
        ],
        out_specs=pl.BlockSpec((batch, blk, dim), lambda i: (0, i, 0)),
        out_shape=jax.ShapeDtypeStruct((batch, seq, dim), x.dtype),
    )(x, w)
